# Initial kernel scaffold; baseline (speedup 1.0000x reference)
import jax
import jax.numpy as jnp
from jax.experimental import pallas as pl


def kernel(x, edge_index, W1, b1, W2, b2):
    raise NotImplementedError("write your pallas kernel here")


# speedup vs baseline: 12.3601x; 12.3601x over previous
import functools

import jax
import jax.numpy as jnp
from jax import lax
from jax.experimental import pallas as pl
from jax.experimental.pallas import tpu as pltpu
from jax.experimental.pallas import tpu_sc as plsc

N = 10000
D = 128
E = 320000

NC = 2
NS = 16
NW = NC * NS

CH = 128
NCHUNK = 79
EPT = NCHUNK * CH
EPAD = EPT * NW
ACC_ROWS = 10112
NPT = ACC_ROWS // NS
ZFULL = NPT // CH
ZREM = NPT - ZFULL * CH

R = 400
G = N // R


def _sc_mesh():
    return plsc.VectorSubcoreMesh(core_axis_name="c", subcore_axis_name="s")


@functools.partial(
    pl.kernel,
    mesh=_sc_mesh(),
    out_type=jax.ShapeDtypeStruct((NC, ACC_ROWS, 16), jnp.float32),
    scratch_types=[
        pltpu.VMEM((NCHUNK, CH), jnp.int32),
        pltpu.VMEM((CH, 16), jnp.float32),
        pltpu.VMEM_SHARED((ACC_ROWS, 16), jnp.float32),
    ],
)
def _deg_kernel(dst_hbm, out_hbm, didx, ones_v, acc):
    c = lax.axis_index("c")
    s = lax.axis_index("s")
    w = c * NS + s

    def fill(r, val):
        ones_v[r, :] = jnp.full((16,), val, jnp.float32)
        return val

    lax.fori_loop(0, CH, fill, 0.0)
    for k in range(ZFULL):
        pltpu.sync_copy(ones_v, acc.at[pl.ds(s * NPT + k * CH, CH)])
    pltpu.sync_copy(ones_v.at[pl.ds(0, ZREM)],
                    acc.at[pl.ds(s * NPT + ZFULL * CH, ZREM)])
    lax.fori_loop(0, CH, fill, 1.0)
    pltpu.sync_copy(dst_hbm.at[w], didx)
    plsc.subcore_barrier()

    def edge_body(j, _):
        pltpu.sync_copy(ones_v, acc.at[didx.at[j]], add=True)
        return 0

    lax.fori_loop(0, NCHUNK, edge_body, 0)
    plsc.subcore_barrier()

    pltpu.sync_copy(acc.at[pl.ds(s * NPT, NPT)], out_hbm.at[c, pl.ds(s * NPT, NPT)])


@functools.partial(
    pl.kernel,
    mesh=_sc_mesh(),
    out_type=jax.ShapeDtypeStruct((NC, ACC_ROWS, D), jnp.float32),
    scratch_types=[
        pltpu.VMEM((NCHUNK, CH), jnp.int32),
        pltpu.VMEM((NCHUNK, CH), jnp.int32),
        pltpu.VMEM((CH, D), jnp.float32),
        pltpu.VMEM_SHARED((ACC_ROWS, D), jnp.float32),
        pltpu.SemaphoreType.DMA,
    ],
)
def _agg_kernel(p_hbm, src_hbm, dst_hbm, out_hbm, sidx, didx, rows, acc, sem):
    c = lax.axis_index("c")
    s = lax.axis_index("s")
    w = c * NS + s

    zero = jnp.zeros((16,), jnp.float32)

    def zb(r, _):
        for kk in range(D // 16):
            rows[r, pl.ds(kk * 16, 16)] = zero
        return 0

    lax.fori_loop(0, CH, zb, 0)

    for k in range(ZFULL):
        pltpu.sync_copy(rows, acc.at[pl.ds(s * NPT + k * CH, CH)])
    pltpu.sync_copy(rows.at[pl.ds(0, ZREM)],
                    acc.at[pl.ds(s * NPT + ZFULL * CH, ZREM)])
    pltpu.sync_copy(src_hbm.at[w], sidx)
    pltpu.sync_copy(dst_hbm.at[w], didx)
    plsc.subcore_barrier()

    def edge_body(j, _):
        pltpu.async_copy(p_hbm.at[sidx.at[j]], rows, sem).wait()
        pltpu.sync_copy(rows, acc.at[didx.at[j]], add=True)
        return 0

    lax.fori_loop(0, NCHUNK, edge_body, 0)
    plsc.subcore_barrier()

    pltpu.sync_copy(acc.at[pl.ds(s * NPT, NPT)], out_hbm.at[c, pl.ds(s * NPT, NPT)])


def _dis(dp):
    deg = 1.0 + dp[0, :, 0] + dp[1, :, 0]
    return lax.rsqrt(deg)[:, None]


def _tc1_body(x_ref, w_ref, dp_ref, o_ref):
    o_ref[...] = _dis(dp_ref[...]) * jnp.dot(
        x_ref[...], w_ref[...], preferred_element_type=jnp.float32)


def _tc2_body(ag_ref, p_ref, b_ref, w_ref, dp_ref, o_ref):
    dis = _dis(dp_ref[...])
    z = dis * (ag_ref[0] + ag_ref[1] + p_ref[...]) + b_ref[...]
    h = jnp.maximum(z, 0.0)
    o_ref[...] = dis * jnp.dot(h, w_ref[...], preferred_element_type=jnp.float32)


def _tc3_body(ag_ref, p_ref, b_ref, dp_ref, o_ref):
    dis = _dis(dp_ref[...])
    z = dis * (ag_ref[0] + ag_ref[1] + p_ref[...]) + b_ref[...]
    m = jnp.max(z, axis=1, keepdims=True)
    lse = jnp.log(jnp.sum(jnp.exp(z - m), axis=1, keepdims=True))
    o_ref[...] = z - m - lse


_blk = pl.BlockSpec((R, D), lambda i: (i, 0))
_wblk = pl.BlockSpec((D, D), lambda i: (0, 0))
_bblk = pl.BlockSpec((1, D), lambda i: (0, 0))
_dpblk = pl.BlockSpec((2, R, 16), lambda i: (0, i, 0))
_agblk = pl.BlockSpec((2, R, D), lambda i: (0, i, 0))
_oshape = jax.ShapeDtypeStruct((N, D), jnp.float32)


def _tc1(x, W, degp):
    return pl.pallas_call(
        _tc1_body, grid=(G,),
        in_specs=[_blk, _wblk, _dpblk],
        out_specs=_blk, out_shape=_oshape)(x, W, degp)


def _tc2(ag, p, b, W, degp):
    return pl.pallas_call(
        _tc2_body, grid=(G,),
        in_specs=[_agblk, _blk, _bblk, _wblk, _dpblk],
        out_specs=_blk, out_shape=_oshape)(ag, p, b, W, degp)


def _tc3(ag, p, b, degp):
    return pl.pallas_call(
        _tc3_body, grid=(G,),
        in_specs=[_agblk, _blk, _bblk, _dpblk],
        out_specs=_blk, out_shape=_oshape)(ag, p, b, degp)


def kernel(x, edge_index, W1, b1, W2, b2):
    ei = edge_index.astype(jnp.int32)
    pad = EPAD - E
    src_p = jnp.concatenate([ei[0], jnp.zeros((pad,), jnp.int32)])
    dst_p = jnp.concatenate([ei[1], jnp.full((pad,), N, jnp.int32)])
    src2d = src_p.reshape(NW, NCHUNK, CH)
    dst2d = dst_p.reshape(NW, NCHUNK, CH)

    degp = _deg_kernel(dst2d)
    p1 = _tc1(x, W1, degp)
    ag1 = _agg_kernel(p1, src2d, dst2d)
    p2 = _tc2(ag1, p1, b1.reshape(1, D), W2, degp)
    ag2 = _agg_kernel(p2, src2d, dst2d)
    return _tc3(ag2, p2, b2.reshape(1, D), degp)
